# baseline (speedup 1.0000x reference)
import functools
import math

import jax
import jax.numpy as jnp
from jax import lax
from jax.experimental import pallas as pl
from jax.experimental.pallas import tpu as pltpu
from jax.experimental.pallas import tpu_sc as plsc

_SMOOTHING = 0.1
_CONFIDENCE = 1.0 - _SMOOTHING
_PAD = 0

_NC = 2
_NS = 16
_NW = _NC * _NS
_LANES = 16
_N_SC = 1024
_GR = 8
_CQ = 4096
_UNROLL = 8


def _sc_build(n_rows, n_cols):
    rows_w = _N_SC // _NW
    groups = rows_w // _GR
    nq = n_cols // _CQ
    iters = groups * nq
    mesh = plsc.VectorSubcoreMesh(core_axis_name="c", subcore_axis_name="s")

    @functools.partial(
        pl.kernel,
        out_type=jax.ShapeDtypeStruct((_NW, _GR, 128), jnp.float32),
        mesh=mesh,
        compiler_params=pltpu.CompilerParams(use_tc_tiling_on_sc=True),
        scratch_types=[
            pltpu.VMEM((rows_w + _LANES,), jnp.int32),
            pltpu.VMEM((_GR, _CQ), jnp.float32),
            pltpu.VMEM((_GR, _CQ), jnp.float32),
            pltpu.VMEM((_GR, 128), jnp.float32),
            pltpu.SemaphoreType.DMA,
            pltpu.SemaphoreType.DMA,
        ],
    )
    def sc_body(x_hbm, tgt_hbm, out_hbm, tgt_v, buf0, buf1, row_v,
                sem_0, sem_1):
        fill = _SMOOTHING / (n_cols - 2)
        c1 = _CONFIDENCE * math.log(_CONFIDENCE) + _SMOOTHING * math.log(fill)
        wid = lax.axis_index("s") * _NC + lax.axis_index("c")
        row0 = wid * rows_w
        bufs = (buf0, buf1)
        sems = (sem_0, sem_1)
        zero16 = jnp.zeros((_LANES,), jnp.float32)
        e0 = jnp.where(lax.iota(jnp.int32, _LANES) == 0, 1.0, 0.0)
        lane_iota = lax.iota(jnp.int32, _LANES)

        def _start(d, b):
            grp = d // nq
            q = d % nq
            pltpu.async_copy(
                x_hbm.at[pl.ds(row0 + grp * _GR, _GR), pl.ds(q * _CQ, _CQ)],
                bufs[b], sems[b])

        for b in range(2):
            _start(b, b)

        pltpu.sync_copy(tgt_hbm.at[pl.ds(row0, rows_w)],
                        tgt_v.at[pl.ds(0, rows_w)])
        tgt_v[pl.ds(rows_w, _LANES)] = jnp.zeros((_LANES,), jnp.int32)

        def outer(k, carry):
            acc_a, acc_0, acc_c, acc_g = carry
            for b in range(2):
                d = 2 * k + b
                grp = d // nq
                q = d % nq
                buf = bufs[b]
                pltpu.make_async_copy(
                    x_hbm.at[pl.ds(0, _GR), pl.ds(0, _CQ)], buf,
                    sems[b]).wait()
                for r in range(_GR):
                    l = grp * _GR + r

                    def chunk(i, acc):
                        base = i * (_LANES * _UNROLL)
                        for j in range(_UNROLL):
                            acc = acc + buf[r, pl.ds(base + j * _LANES,
                                                     _LANES)]
                        return acc

                    acc_row = lax.fori_loop(
                        0, _CQ // (_LANES * _UNROLL), chunk, zero16)
                    tvec = tgt_v[pl.ds(l, _LANES)]
                    t_i = tvec[0]
                    m = jnp.where(t_i != _PAD, 1.0, 0.0)
                    off = t_i - q * _CQ
                    in_q = jnp.where((off >= 0) & (off < _CQ), m, 0.0)
                    offc = jnp.clip(off, 0, _CQ - 1)
                    gchunk = buf[r, pl.ds((offc // _LANES) * _LANES, _LANES)]
                    gvec = jnp.where(lane_iota == offc % _LANES, gchunk, 0.0)
                    first = jnp.where(q == 0, m, 0.0)
                    acc_a = acc_a + m * acc_row
                    acc_0 = acc_0 + (first * buf[r, pl.ds(0, _LANES)][0]) * e0
                    acc_c = acc_c + first * e0
                    acc_g = acc_g + in_q * gvec

                @pl.when(d + 2 < iters)
                def _next():
                    _start(d + 2, b)
            return (acc_a, acc_0, acc_c, acc_g)

        acc_a, acc_0, acc_c, acc_g = lax.fori_loop(
            0, iters // 2, outer, (zero16, zero16, zero16, zero16))

        dvec = (c1 * acc_c - fill * (acc_a - acc_0)
                - (_CONFIDENCE - fill) * acc_g)
        for rr in range(_GR):
            for j in range(128 // _LANES):
                row_v[rr, pl.ds(j * _LANES, _LANES)] = zero16
        row_v[0, pl.ds(0, _LANES)] = dvec
        pltpu.sync_copy(row_v, out_hbm.at[wid])

    return sc_body


def _tc_body(fill, c1, x_ref, t_ref, out_ref):
    i = pl.program_id(0)
    xb = x_ref[...]
    t = t_ref[0, 0, :]
    row_sum = jnp.sum(xb, axis=1)
    col0 = xb[:, 0]
    col_iota = lax.broadcasted_iota(jnp.int32, xb.shape, 1)
    gathered = jnp.sum(jnp.where(col_iota == t[:, None], xb, 0.0), axis=1)
    contrib = jnp.where(
        t != _PAD,
        c1 - fill * (row_sum - col0) - (_CONFIDENCE - fill) * gathered,
        0.0)
    partial = jnp.sum(contrib)

    @pl.when(i == 0)
    def _init():
        out_ref[0, 0] = 0.0

    out_ref[0, 0] += partial


def _combine_body(a_ref, g_ref, out_ref):
    out_ref[0, 0] = a_ref[0, 0] + jnp.sum(g_ref[...])


def kernel(x, target):
    n, v = x.shape
    fill = _SMOOTHING / (v - 2)
    c1 = _CONFIDENCE * math.log(_CONFIDENCE) + _SMOOTHING * math.log(fill)
    t32 = target.astype(jnp.int32)

    sc_partials = _sc_build(n, v)(x, t32)

    r = 256
    blk0 = _N_SC // r
    n_blocks = (n - _N_SC) // r
    tc_part = pl.pallas_call(
        functools.partial(_tc_body, fill, c1),
        grid=(n_blocks,),
        in_specs=[
            pl.BlockSpec((r, v), lambda i: (i + blk0, 0)),
            pl.BlockSpec((1, 1, r), lambda i: (i + blk0, 0, 0)),
        ],
        out_specs=pl.BlockSpec(memory_space=pltpu.SMEM),
        out_shape=jax.ShapeDtypeStruct((1, 1), jnp.float32),
        compiler_params=pltpu.CompilerParams(
            dimension_semantics=("arbitrary",)),
    )(x, t32.reshape(n // r, 1, r))

    total = pl.pallas_call(
        _combine_body,
        in_specs=[
            pl.BlockSpec(memory_space=pltpu.SMEM),
            pl.BlockSpec((_NW, _GR, 128), lambda: (0, 0, 0)),
        ],
        out_specs=pl.BlockSpec(memory_space=pltpu.SMEM),
        out_shape=jax.ShapeDtypeStruct((1, 1), jnp.float32),
    )(tc_part, sc_partials)
    return total[0, 0]


# speedup vs baseline: 1.0074x; 1.0074x over previous
import functools
import math

import jax
import jax.numpy as jnp
from jax import lax
from jax.experimental import pallas as pl
from jax.experimental.pallas import tpu as pltpu
from jax.experimental.pallas import tpu_sc as plsc

_SMOOTHING = 0.1
_CONFIDENCE = 1.0 - _SMOOTHING
_PAD = 0

_NC = 2
_NS = 16
_NW = _NC * _NS
_LANES = 16
_N_SC = 256
_GR = 8
_CQ = 4096
_UNROLL = 8


def _sc_build(n_rows, n_cols):
    rows_w = _N_SC // _NW
    groups = rows_w // _GR
    nq = n_cols // _CQ
    iters = groups * nq
    mesh = plsc.VectorSubcoreMesh(core_axis_name="c", subcore_axis_name="s")

    @functools.partial(
        pl.kernel,
        out_type=jax.ShapeDtypeStruct((_NW, _GR, 128), jnp.float32),
        mesh=mesh,
        compiler_params=pltpu.CompilerParams(use_tc_tiling_on_sc=True),
        scratch_types=[
            pltpu.VMEM((rows_w + _LANES,), jnp.int32),
            pltpu.VMEM((_GR, _CQ), jnp.float32),
            pltpu.VMEM((_GR, _CQ), jnp.float32),
            pltpu.VMEM((_GR, 128), jnp.float32),
            pltpu.SemaphoreType.DMA,
            pltpu.SemaphoreType.DMA,
        ],
    )
    def sc_body(x_hbm, tgt_hbm, out_hbm, tgt_v, buf0, buf1, row_v,
                sem_0, sem_1):
        fill = _SMOOTHING / (n_cols - 2)
        c1 = _CONFIDENCE * math.log(_CONFIDENCE) + _SMOOTHING * math.log(fill)
        wid = lax.axis_index("s") * _NC + lax.axis_index("c")
        row0 = wid * rows_w
        bufs = (buf0, buf1)
        sems = (sem_0, sem_1)
        zero16 = jnp.zeros((_LANES,), jnp.float32)
        e0 = jnp.where(lax.iota(jnp.int32, _LANES) == 0, 1.0, 0.0)
        lane_iota = lax.iota(jnp.int32, _LANES)

        def _start(d, b):
            grp = d // nq
            q = d % nq
            pltpu.async_copy(
                x_hbm.at[pl.ds(row0 + grp * _GR, _GR), pl.ds(q * _CQ, _CQ)],
                bufs[b], sems[b])

        for b in range(2):
            _start(b, b)

        pltpu.sync_copy(tgt_hbm.at[pl.ds(row0, rows_w)],
                        tgt_v.at[pl.ds(0, rows_w)])
        tgt_v[pl.ds(rows_w, _LANES)] = jnp.zeros((_LANES,), jnp.int32)

        def outer(k, carry):
            acc_a, acc_0, acc_c, acc_g = carry
            for b in range(2):
                d = 2 * k + b
                grp = d // nq
                q = d % nq
                buf = bufs[b]
                pltpu.make_async_copy(
                    x_hbm.at[pl.ds(0, _GR), pl.ds(0, _CQ)], buf,
                    sems[b]).wait()
                for r in range(_GR):
                    l = grp * _GR + r

                    def chunk(i, acc):
                        base = i * (_LANES * _UNROLL)
                        for j in range(_UNROLL):
                            acc = acc + buf[r, pl.ds(base + j * _LANES,
                                                     _LANES)]
                        return acc

                    acc_row = lax.fori_loop(
                        0, _CQ // (_LANES * _UNROLL), chunk, zero16)
                    tvec = tgt_v[pl.ds(l, _LANES)]
                    t_i = tvec[0]
                    m = jnp.where(t_i != _PAD, 1.0, 0.0)
                    off = t_i - q * _CQ
                    in_q = jnp.where((off >= 0) & (off < _CQ), m, 0.0)
                    offc = jnp.clip(off, 0, _CQ - 1)
                    gchunk = buf[r, pl.ds((offc // _LANES) * _LANES, _LANES)]
                    gvec = jnp.where(lane_iota == offc % _LANES, gchunk, 0.0)
                    first = jnp.where(q == 0, m, 0.0)
                    acc_a = acc_a + m * acc_row
                    acc_0 = acc_0 + (first * buf[r, pl.ds(0, _LANES)][0]) * e0
                    acc_c = acc_c + first * e0
                    acc_g = acc_g + in_q * gvec

                @pl.when(d + 2 < iters)
                def _next():
                    _start(d + 2, b)
            return (acc_a, acc_0, acc_c, acc_g)

        acc_a, acc_0, acc_c, acc_g = lax.fori_loop(
            0, iters // 2, outer, (zero16, zero16, zero16, zero16))

        dvec = (c1 * acc_c - fill * (acc_a - acc_0)
                - (_CONFIDENCE - fill) * acc_g)
        for rr in range(_GR):
            for j in range(128 // _LANES):
                row_v[rr, pl.ds(j * _LANES, _LANES)] = zero16
        row_v[0, pl.ds(0, _LANES)] = dvec
        pltpu.sync_copy(row_v, out_hbm.at[wid])

    return sc_body


def _tc_body(fill, c1, x_ref, t_ref, out_ref):
    i = pl.program_id(0)
    xb = x_ref[...]
    t = t_ref[0, 0, :]
    row_sum = jnp.sum(xb, axis=1)
    col0 = xb[:, 0]
    col_iota = lax.broadcasted_iota(jnp.int32, xb.shape, 1)
    gathered = jnp.sum(jnp.where(col_iota == t[:, None], xb, 0.0), axis=1)
    contrib = jnp.where(
        t != _PAD,
        c1 - fill * (row_sum - col0) - (_CONFIDENCE - fill) * gathered,
        0.0)
    partial = jnp.sum(contrib)

    @pl.when(i == 0)
    def _init():
        out_ref[0, 0] = 0.0

    out_ref[0, 0] += partial


def _combine_body(a_ref, g_ref, out_ref):
    out_ref[0, 0] = a_ref[0, 0] + jnp.sum(g_ref[...])


def kernel(x, target):
    n, v = x.shape
    fill = _SMOOTHING / (v - 2)
    c1 = _CONFIDENCE * math.log(_CONFIDENCE) + _SMOOTHING * math.log(fill)
    t32 = target.astype(jnp.int32)

    sc_partials = _sc_build(n, v)(x, t32)

    r = 256
    blk0 = _N_SC // r
    n_blocks = (n - _N_SC) // r
    tc_part = pl.pallas_call(
        functools.partial(_tc_body, fill, c1),
        grid=(n_blocks,),
        in_specs=[
            pl.BlockSpec((r, v), lambda i: (i + blk0, 0)),
            pl.BlockSpec((1, 1, r), lambda i: (i + blk0, 0, 0)),
        ],
        out_specs=pl.BlockSpec(memory_space=pltpu.SMEM),
        out_shape=jax.ShapeDtypeStruct((1, 1), jnp.float32),
        compiler_params=pltpu.CompilerParams(
            dimension_semantics=("arbitrary",)),
    )(x, t32.reshape(n // r, 1, r))

    total = pl.pallas_call(
        _combine_body,
        in_specs=[
            pl.BlockSpec(memory_space=pltpu.SMEM),
            pl.BlockSpec((_NW, _GR, 128), lambda: (0, 0, 0)),
        ],
        out_specs=pl.BlockSpec(memory_space=pltpu.SMEM),
        out_shape=jax.ShapeDtypeStruct((1, 1), jnp.float32),
    )(tc_part, sc_partials)
    return total[0, 0]
